# baseline (speedup 1.0000x reference)
import jax
import jax.numpy as jnp
from jax import lax
from jax.experimental import pallas as pl
from jax.experimental.pallas import tpu as pltpu
from jax.experimental.pallas import tpu_sc as plsc

DIM_OUT = 64
EMB = 128
E = 320000

NUM_CORES = 2
NUM_SUBCORES = 16
NUM_WORKERS = NUM_CORES * NUM_SUBCORES
LANES = 16
STRIPES = EMB // LANES

PER_WORKER = E // NUM_WORKERS
BLOCK = 80
NBLK = PER_WORKER // BLOCK
GROUPS = BLOCK // LANES
TBL = (DIM_OUT + 1) * EMB
RING = 5
RSTEPS = NBLK // RING


def _do_group(g, table_v, buf_v, meta_v):
    off = g * LANES
    labs = meta_v[pl.ds(off, LANES)]
    sp = meta_v[pl.ds(BLOCK + off, LANES)]
    dr = meta_v[pl.ds(2 * BLOCK + off, LANES)]
    keep = sp * (1 - dr)
    labs = labs * keep + DIM_OUT * (1 - keep)
    lane = lax.broadcasted_iota(jnp.int32, (LANES,), 0)
    lab128 = labs << 7
    row128 = (lane + off) << 7
    def gathers(u):
        gidx = lab128 | ((lane + u) & (LANES - 1))
        return [plsc.load_gather(
                    table_v.at[pl.ds(s * LANES, TBL - s * LANES)], [gidx])
                for s in range(STRIPES)]

    def scatters(u, vals):
        sidx = row128 | ((lane + u) & (LANES - 1))
        for s in range(STRIPES):
            plsc.addupdate_scatter(
                buf_v.at[pl.ds(s * LANES, BLOCK * EMB - s * LANES)],
                [sidx], vals[s])

    vals = gathers(0)
    for u in range(1, LANES):
        nxt = gathers(u)
        scatters(u - 1, vals)
        vals = nxt
    scatters(LANES - 1, vals)
    return 0


def _body(edge_hbm, wb_hbm, meta_hbm, out_hbm, table_v, *scr):
    bufs = scr[0:RING]
    metas = scr[RING:2 * RING]
    sin = scr[2 * RING:3 * RING]
    sout = scr[3 * RING:4 * RING]

    wid = lax.axis_index("s") * NUM_CORES + lax.axis_index("c")
    base_w = wid * PER_WORKER

    pltpu.sync_copy(wb_hbm, table_v)

    def start_in(n, k):
        base = (base_w + n * BLOCK) * EMB
        gblk = wid * NBLK + n
        pltpu.async_copy(edge_hbm.at[pl.ds(base, BLOCK * EMB)],
                         bufs[k], sin[k])
        pltpu.async_copy(meta_hbm.at[pl.ds(gblk * 3 * BLOCK, 3 * BLOCK)],
                         metas[k], sin[k])

    def wait_in(k):
        pltpu.make_async_copy(edge_hbm.at[pl.ds(0, BLOCK * EMB)],
                              bufs[k], sin[k]).wait()
        pltpu.make_async_copy(meta_hbm.at[pl.ds(0, 3 * BLOCK)],
                              metas[k], sin[k]).wait()

    def start_out(n, k):
        base = (base_w + n * BLOCK) * EMB
        pltpu.async_copy(bufs[k], out_hbm.at[pl.ds(base, BLOCK * EMB)],
                         sout[k])

    def wait_out(k):
        pltpu.make_async_copy(bufs[k], out_hbm.at[pl.ds(0, BLOCK * EMB)],
                              sout[k]).wait()

    for k in range(RING - 2):
        start_in(k, k)

    def ring_turn(t, _):
        for k in range(RING):
            n = t * RING + k
            kp = (k + RING - 2) % RING
            @pl.when(n + RING - 2 < NBLK)
            def _():
                @pl.when(n >= 2)
                def _():
                    wait_out(kp)
                start_in(n + RING - 2, kp)
            wait_in(k)
            for g in range(GROUPS):
                _do_group(g, table_v, bufs[k], metas[k])
            start_out(n, k)
        return 0

    lax.fori_loop(0, RSTEPS, ring_turn, 0)
    for k in range(RING):
        wait_out(k)


@jax.jit
def _run(edge_flat, wb_flat, meta_flat):
    mesh = plsc.VectorSubcoreMesh(
        core_axis_name="c", subcore_axis_name="s",
        num_cores=NUM_CORES, num_subcores=NUM_SUBCORES)
    return pl.kernel(
        _body,
        out_type=jax.ShapeDtypeStruct((E * EMB,), jnp.float32),
        mesh=mesh,
        compiler_params=pltpu.CompilerParams(needs_layout_passes=False),
        scratch_types=(
            [pltpu.VMEM((TBL,), jnp.float32)]
            + [pltpu.VMEM((BLOCK * EMB,), jnp.float32)] * RING
            + [pltpu.VMEM((3 * BLOCK,), jnp.int32)] * RING
            + [pltpu.SemaphoreType.DMA] * (2 * RING)
        ),
    )(edge_flat, wb_flat, meta_flat)


def kernel(edge_attr, W, b, label, split_mask, drop_mask):
    wb_flat = (W + b[None, :]).reshape(-1)
    meta_flat = jnp.stack(
        [jnp.asarray(label, jnp.int32),
         split_mask.astype(jnp.int32),
         drop_mask.astype(jnp.int32)], axis=0) \
        .reshape(3, E // BLOCK, BLOCK).transpose(1, 0, 2).reshape(-1)
    out = _run(edge_attr.reshape(-1), wb_flat, meta_flat)
    return out.reshape(E, EMB)


# speedup vs baseline: 2.1070x; 2.1070x over previous
import jax
import jax.numpy as jnp
from jax import lax
from jax.experimental import pallas as pl
from jax.experimental.pallas import tpu as pltpu
from jax.experimental.pallas import tpu_sc as plsc

DIM_OUT = 64
EMB = 128
E = 320000

NUM_CORES = 2
NUM_SUBCORES = 16
NUM_WORKERS = NUM_CORES * NUM_SUBCORES
LANES = 16
STRIPES = EMB // LANES
ROWW = EMB

PER_WORKER = E // NUM_WORKERS
BLOCK = 400
NBLK = PER_WORKER // BLOCK
GROUPS = BLOCK // LANES
TBL = (DIM_OUT + 1) * EMB


def _do_group(g, table_v, buf_v, meta_v):
    off = g * LANES
    labs = meta_v[pl.ds(off, LANES)]
    sp = meta_v[pl.ds(BLOCK + off, LANES)]
    dr = meta_v[pl.ds(2 * BLOCK + off, LANES)]
    keep = sp * (1 - dr)
    labs = labs * keep + DIM_OUT * (1 - keep)
    lane = lax.broadcasted_iota(jnp.int32, (LANES,), 0)
    lab128 = labs << 7
    row128 = (lane + off) << 7
    def gathers(u):
        gidx = lab128 | ((lane + u) & (LANES - 1))
        return [plsc.load_gather(
                    table_v.at[pl.ds(s * LANES, TBL - s * LANES)], [gidx])
                for s in range(STRIPES)]

    def scatters(u, vals):
        sidx = row128 | ((lane + u) & (LANES - 1))
        for s in range(STRIPES):
            plsc.addupdate_scatter(
                buf_v.at[pl.ds(s * LANES, BLOCK * EMB - s * LANES)],
                [sidx], vals[s])

    vals = gathers(0)
    for u in range(1, LANES):
        nxt = gathers(u)
        scatters(u - 1, vals)
        vals = nxt
    scatters(LANES - 1, vals)
    return 0


def _body(edge_hbm, wb_hbm, meta_hbm, out_hbm, table_v,
          buf0, buf1, meta0, meta1, sin0, sin1, sout0, sout1):
    bufs = (buf0, buf1)
    metas = (meta0, meta1)
    sin = (sin0, sin1)
    sout = (sout0, sout1)

    wid = lax.axis_index("s") * NUM_CORES + lax.axis_index("c")
    base_w = wid * PER_WORKER

    pltpu.sync_copy(wb_hbm, table_v)

    def start_in(n, k):
        base = (base_w + n * BLOCK) * EMB
        gblk = wid * NBLK + n
        pltpu.async_copy(edge_hbm.at[pl.ds(base, BLOCK * EMB)],
                         bufs[k], sin[k])
        pltpu.async_copy(meta_hbm.at[pl.ds(gblk * 3 * BLOCK, 3 * BLOCK)],
                         metas[k], sin[k])

    def wait_in(k):
        pltpu.make_async_copy(edge_hbm.at[pl.ds(0, BLOCK * EMB)],
                              bufs[k], sin[k]).wait()
        pltpu.make_async_copy(meta_hbm.at[pl.ds(0, 3 * BLOCK)],
                              metas[k], sin[k]).wait()

    def start_out(n, k):
        base = (base_w + n * BLOCK) * EMB
        pltpu.async_copy(bufs[k], out_hbm.at[pl.ds(base, BLOCK * EMB)],
                         sout[k])

    def wait_out(k):
        pltpu.make_async_copy(bufs[k], out_hbm.at[pl.ds(0, BLOCK * EMB)],
                              sout[k]).wait()

    start_in(0, 0)

    def step(n, _):
        def on_buf(k):
            @pl.when(lax.rem(n, 2) == k)
            def _():
                wait_in(k)
                @pl.when(n >= 1)
                def _():
                    wait_out(1 - k)

                @pl.when(n + 1 < NBLK)
                def _():
                    start_in(n + 1, 1 - k)

                lax.fori_loop(
                    0, GROUPS,
                    lambda g, c: _do_group(g, table_v, bufs[k], metas[k]),
                    0)
                start_out(n, k)

        on_buf(0)
        on_buf(1)
        return 0

    lax.fori_loop(0, NBLK, step, 0)
    wait_out((NBLK - 1) % 2)


@jax.jit
def _run(edge_flat, wb_flat, meta_flat):
    mesh = plsc.VectorSubcoreMesh(
        core_axis_name="c", subcore_axis_name="s",
        num_cores=NUM_CORES, num_subcores=NUM_SUBCORES)
    return pl.kernel(
        _body,
        out_type=jax.ShapeDtypeStruct((E * EMB,), jnp.float32),
        mesh=mesh,
        compiler_params=pltpu.CompilerParams(needs_layout_passes=False),
        scratch_types=(
            [pltpu.VMEM((TBL,), jnp.float32)]
            + [pltpu.VMEM((BLOCK * EMB,), jnp.float32)] * 2
            + [pltpu.VMEM((3 * BLOCK,), jnp.int32)] * 2
            + [pltpu.SemaphoreType.DMA] * 4
        ),
    )(edge_flat, wb_flat, meta_flat)


def kernel(edge_attr, W, b, label, split_mask, drop_mask):
    wb_flat = (W + b[None, :]).reshape(-1)
    meta_flat = jnp.stack(
        [jnp.asarray(label, jnp.int32),
         split_mask.astype(jnp.int32),
         drop_mask.astype(jnp.int32)], axis=0) \
        .reshape(3, E // BLOCK, BLOCK).transpose(1, 0, 2).reshape(-1)
    out = _run(edge_attr.reshape(-1), wb_flat, meta_flat)
    return out.reshape(E, EMB)
